# Initial kernel scaffold; baseline (speedup 1.0000x reference)
import jax
import jax.numpy as jnp
from jax.experimental import pallas as pl


def kernel(inputs):
    raise NotImplementedError("write your pallas kernel here")


# speedup vs baseline: 5.4510x; 5.4510x over previous
import jax
import jax.numpy as jnp
from jax.experimental import pallas as pl
from jax.experimental.pallas import tpu as pltpu

K = 64
LANES = 128
CHUNK = 512


def _ce_small(v, d, size, flip):
    rows = v.shape[0]
    ii = jax.lax.broadcasted_iota(jnp.int32, v.shape, 0)
    low_bit = (ii & d) == 0
    asc_blk = (ii & size) != 0
    partner = jnp.where(low_bit, pltpu.roll(v, rows - d, 0), pltpu.roll(v, d, 0))
    want_max = (low_bit != asc_blk) != flip
    return jnp.where(want_max, jnp.maximum(v, partner), jnp.minimum(v, partner))


def _ce_big(v, d, size, flip):
    lanes = v.shape[1]
    g = v.reshape(-1, 2 * d, lanes)
    a = g[:, :d, :]
    b = g[:, d:, :]
    hi = jnp.maximum(a, b)
    lo = jnp.minimum(a, b)
    m = size // (2 * d)
    gi = jax.lax.broadcasted_iota(jnp.int32, (g.shape[0], 1, 1), 0)
    asc = ((gi & m) != 0) != flip
    top = jnp.where(asc, lo, hi)
    bot = jnp.where(asc, hi, lo)
    return jnp.concatenate([top, bot], axis=1).reshape(-1, lanes)


def _ce(v, d, size, flip=False):
    if d >= 8:
        return _ce_big(v, d, size, flip)
    return _ce_small(v, d, size, flip)


def _resort64(v, flip):
    for d in (32, 16, 8, 4, 2, 1):
        v = _ce(v, d, K, flip)
    return v


def _chunk_topk_asc(v):
    size = 2
    while size <= K:
        d = size // 2
        while d >= 1:
            v = _ce(v, d, size)
            d //= 2
        size *= 2
    while v.shape[0] > K:
        g = v.reshape(-1, 2 * K, v.shape[1])
        v = jnp.maximum(g[:, :K, :], g[:, K:, :]).reshape(-1, v.shape[1])
        v = _resort64(v, flip=(v.shape[0] == K))
    return v


def _kmax_body(x_ref, o_ref):
    s_idx = pl.program_id(2)
    chunk = _chunk_topk_asc(x_ref[0])

    @pl.when(s_idx == 0)
    def _init():
        o_ref[0] = jnp.full((K, LANES), -jnp.inf, jnp.float32)

    acc = o_ref[0]
    merged = jnp.maximum(acc, chunk)
    o_ref[0] = _resort64(merged, flip=False)


def kernel(inputs):
    b, s, c = inputs.shape
    chunk = min(CHUNK, s)
    grid = (b, c // LANES, s // chunk)
    out = pl.pallas_call(
        _kmax_body,
        grid=grid,
        in_specs=[pl.BlockSpec((1, chunk, LANES), lambda i, j, k: (i, k, j))],
        out_specs=pl.BlockSpec((1, K, LANES), lambda i, j, k: (i, 0, j)),
        out_shape=jax.ShapeDtypeStruct((b, K, c), jnp.float32),
    )(inputs)
    return out
